# baseline (speedup 1.0000x reference)
import functools

import jax
import jax.numpy as jnp
from jax import lax
from jax.experimental import pallas as pl
from jax.experimental.pallas import tpu as pltpu
from jax.experimental.pallas import tpu_sc as plsc

F32 = jnp.float32
I32 = jnp.int32

NC = 2
NS = 16
LN = 16


def _attn_proj(a, asv, adv, S):
    asrc = jnp.dot(a * asv, S, preferred_element_type=F32)
    adst = jnp.dot(a * adv, S, preferred_element_type=F32)
    return asrc, adst


def _tc_in_body(x_ref, w_ref, asv_ref, adv_ref, S_ref,
                h4_ref, asrc_ref, adst_ref, *, n_chunks):
    a = jnp.dot(x_ref[...], w_ref[...], preferred_element_type=F32)
    cw = a.shape[1] // n_chunks
    for ci in range(n_chunks):
        h4_ref[ci] = a[:, ci * cw:(ci + 1) * cw]
    asrc_ref[...], adst_ref[...] = _attn_proj(a, asv_ref[...], adv_ref[...],
                                              S_ref[...])


def _tc_in(x, W, asv, adv, S, n_chunks, bn=1000):
    Nn, K = x.shape
    bn = bn if Nn % bn == 0 else Nn
    HC = W.shape[1]
    H = S.shape[1]
    cw = HC // n_chunks
    return pl.pallas_call(
        functools.partial(_tc_in_body, n_chunks=n_chunks),
        grid=(Nn // bn,),
        in_specs=[pl.BlockSpec((bn, K), lambda i: (i, 0)),
                  pl.BlockSpec((K, HC), lambda i: (0, 0)),
                  pl.BlockSpec((1, HC), lambda i: (0, 0)),
                  pl.BlockSpec((1, HC), lambda i: (0, 0)),
                  pl.BlockSpec((HC, H), lambda i: (0, 0))],
        out_specs=[pl.BlockSpec((n_chunks, bn, cw), lambda i: (0, i, 0)),
                   pl.BlockSpec((bn, H), lambda i: (i, 0)),
                   pl.BlockSpec((bn, H), lambda i: (i, 0))],
        out_shape=[jax.ShapeDtypeStruct((n_chunks, Nn, cw), F32),
                   jax.ShapeDtypeStruct((Nn, H), F32),
                   jax.ShapeDtypeStruct((Nn, H), F32)],
    )(x, W, asv, adv, S)


def _normalize_block(acc, den, n_chunks, cw):
    feats = [acc[ci] / den[:, ci:ci + 1] for ci in range(n_chunks)]
    return jnp.concatenate(feats, axis=1) if len(feats) > 1 else feats[0]


def _tc_mid_body(acc_ref, den_ref, b_ref, w_ref, asv_ref, adv_ref, S_ref,
                 h4_ref, asrc_ref, adst_ref, *, n_chunks, cw, out_chunks):
    feat = _normalize_block(acc_ref[...], den_ref[...], n_chunks, cw)
    xx = jnp.maximum(feat + b_ref[...], 0.0)
    a = jnp.dot(xx, w_ref[...], preferred_element_type=F32)
    ocw = a.shape[1] // out_chunks
    for ci in range(out_chunks):
        h4_ref[ci] = a[:, ci * ocw:(ci + 1) * ocw]
    asrc_ref[...], adst_ref[...] = _attn_proj(a, asv_ref[...], adv_ref[...],
                                              S_ref[...])


def _tc_mid(acc, den, b, W, asv, adv, S, n_chunks, cw, out_chunks, bn=1000):
    Nn = acc.shape[1]
    bn = bn if Nn % bn == 0 else Nn
    HCin = n_chunks * cw
    HCout = W.shape[1]
    H = S.shape[1]
    ocw = HCout // out_chunks
    return pl.pallas_call(
        functools.partial(_tc_mid_body, n_chunks=n_chunks, cw=cw,
                          out_chunks=out_chunks),
        grid=(Nn // bn,),
        in_specs=[pl.BlockSpec((n_chunks, bn, cw), lambda i: (0, i, 0)),
                  pl.BlockSpec((bn, n_chunks), lambda i: (i, 0)),
                  pl.BlockSpec((1, HCin), lambda i: (0, 0)),
                  pl.BlockSpec((HCin, HCout), lambda i: (0, 0)),
                  pl.BlockSpec((1, HCout), lambda i: (0, 0)),
                  pl.BlockSpec((1, HCout), lambda i: (0, 0)),
                  pl.BlockSpec((HCout, H), lambda i: (0, 0))],
        out_specs=[pl.BlockSpec((out_chunks, bn, ocw), lambda i: (0, i, 0)),
                   pl.BlockSpec((bn, H), lambda i: (i, 0)),
                   pl.BlockSpec((bn, H), lambda i: (i, 0))],
        out_shape=[jax.ShapeDtypeStruct((out_chunks, Nn, ocw), F32),
                   jax.ShapeDtypeStruct((Nn, H), F32),
                   jax.ShapeDtypeStruct((Nn, H), F32)],
    )(acc, den, b, W, asv, adv, S)


def _tc_final_body(acc_ref, den_ref, b3_ref, cW1_ref, cb1_ref, cW2_ref,
                   cb2_ref, out_ref, *, cw):
    p = acc_ref[0] + acc_ref[1]
    d = den_ref[:, 0:1] + den_ref[:, 1:2]
    feat = p / d
    h3 = feat + b3_ref[...]
    g = jnp.mean(h3, axis=0, keepdims=True)
    g1 = jnp.maximum(jnp.dot(g, cW1_ref[...],
                             preferred_element_type=F32) + cb1_ref[...], 0.0)
    logits = jnp.dot(g1, cW2_ref[...],
                     preferred_element_type=F32) + cb2_ref[...]
    m = jnp.max(logits, axis=1, keepdims=True)
    ls = m + jnp.log(jnp.sum(jnp.exp(logits - m), axis=1, keepdims=True))
    out_ref[...] = logits - ls


def _tc_final(acc3, den3, b3, cW1, cb1, cW2, cb2, cw):
    return pl.pallas_call(
        functools.partial(_tc_final_body, cw=cw),
        out_shape=jax.ShapeDtypeStruct((1, 2), F32),
    )(acc3, den3, b3, cW1, cb1, cW2, cb2)


def _sc_edge_kernel(Nn, EP, Ereal, n_chunks, cpc, cw, K, SB, epw,
                    split_edges, tabw):
    rpt = Nn // NS
    zr = 5
    nvec = K // LN
    nb_sb = SB // K
    nbp = nb_sb // 2
    nsb = epw // SB
    assert nb_sb % 2 == 0 and epw % SB == 0 and SB % K == 0
    mesh = plsc.VectorSubcoreMesh(core_axis_name="c", subcore_axis_name="s",
                                  num_cores=NC, num_subcores=NS)
    out_rows = NC * cpc

    @functools.partial(
        pl.kernel, mesh=mesh,
        compiler_params=pltpu.CompilerParams(use_tc_tiling_on_sc=False,
                                             needs_layout_passes=False),
        out_type=[jax.ShapeDtypeStruct((out_rows, Nn, cw), F32),
                  jax.ShapeDtypeStruct((out_rows, Nn), F32)],
        scratch_types=[
            pltpu.VMEM_SHARED((Nn, cw), F32),
            pltpu.VMEM_SHARED((NS, Nn), F32),
            pltpu.VMEM((tabw,), F32),
            pltpu.VMEM((tabw,), F32),
            pltpu.VMEM((Nn,), F32),
            pltpu.VMEM((640,), F32),
            pltpu.VMEM((640,), F32),
            pltpu.VMEM((nb_sb, K), I32),
            pltpu.VMEM((nb_sb, K), I32),
            pltpu.VMEM((K,), I32),
            pltpu.VMEM((K,), I32),
            pltpu.VMEM((2, K, cw), F32),
            pltpu.VMEM((2, K, cw), F32),
            pltpu.VMEM((zr, cw), F32),
            pltpu.SemaphoreType.DMA,
            pltpu.SemaphoreType.DMA,
        ],
    )
    def k(h_hbm, src_hbm, dst_hbm, as_hbm, ad_hbm, out_hbm, den_hbm,
          acc, dstage, tabs, tabd, dtab, dstrip, dsum,
          sidx, didx, gidx0, gidx1, rows, sbuf,
          zbuf, sem_g, sem_s):
        c = lax.axis_index("c")
        s = lax.axis_index("s")
        zero16 = jnp.zeros((LN,), F32)
        for r in range(zr):
            for q in range(cw // LN):
                zbuf[r, pl.ds(q * LN, LN)] = zero16
        if split_edges:
            ebase = (c * NS + s) * epw
        else:
            ebase = s * epw
        brow0 = ebase // K
        iota16 = lax.iota(I32, LN)
        gixb = (gidx0, gidx1)

        def issue_gather(b, p, ci):
            if n_chunks > 1:
                def gix(t, carry):
                    sv = sidx[b, pl.ds(t * LN, LN)]
                    gixb[p][pl.ds(t * LN, LN)] = sv + ci * Nn
                    return carry
                lax.fori_loop(0, nvec, gix, 0)
                pltpu.async_copy(h_hbm.at[gixb[p]], rows.at[p], sem_g)
            else:
                pltpu.async_copy(h_hbm.at[sidx.at[b]], rows.at[p], sem_g)

        def wait_gather():
            pltpu.make_async_copy(h_hbm.at[pl.ds(0, K)], rows.at[0],
                                  sem_g).wait()

        def issue_scatter(b, p):
            pltpu.async_copy(sbuf.at[p], acc.at[didx.at[b]], sem_s,
                             add=True)

        def wait_scatter():
            pltpu.make_async_copy(sbuf.at[0], acc.at[pl.ds(0, K)],
                                  sem_s).wait()

        def scale(b, p, e0sb):
            def scale16(t, carry):
                sv = sidx[b, pl.ds(t * LN, LN)]
                dv = didx[b, pl.ds(t * LN, LN)]
                valid = (e0sb + b * K + t * LN + iota16) < Ereal
                gs = plsc.load_gather(tabs, [sv])
                gd = plsc.load_gather(tabd, [dv])
                al = gs + gd
                al = jnp.maximum(al, 0.2 * al)
                wv = jnp.where(valid, jnp.exp(al), 0.0)
                plsc.addupdate_scatter(dtab, [dv], wv)
                for lane in range(LN):
                    kk = t * LN + lane
                    a0 = wv[lane]
                    v0 = jnp.full((LN,), a0, F32)
                    for q in range(cw // LN):
                        sbuf[p, kk, pl.ds(q * LN, LN)] = (
                            rows[p, kk, pl.ds(q * LN, LN)] * v0)
                return carry
            lax.fori_loop(0, nvec, scale16, 0)

        def chunk_body(j, carry):
            ci = c * cpc + j
            for z in range(rpt // zr):
                pltpu.sync_copy(zbuf, acc.at[pl.ds(s * rpt + z * zr, zr)])

            def dzero(r, carry2):
                dtab[pl.ds(r * LN, LN)] = zero16
                return carry2
            lax.fori_loop(0, Nn // LN, dzero, 0)
            plsc.subcore_barrier()
            if n_chunks > 1:
                pltpu.sync_copy(as_hbm.at[ci], tabs)
                pltpu.sync_copy(ad_hbm.at[ci], tabd)
            else:
                pltpu.sync_copy(as_hbm, tabs)
                pltpu.sync_copy(ad_hbm, tabd)

            def sb_body(q, carry2):
                row0 = brow0 + q * nb_sb
                pltpu.sync_copy(src_hbm.at[pl.ds(row0, nb_sb)], sidx)
                pltpu.sync_copy(dst_hbm.at[pl.ds(row0, nb_sb)], didx)
                e0sb = ebase + q * SB
                issue_gather(0, 0, ci)
                issue_gather(1, 1, ci)
                for p in range(2):
                    wait_gather()
                    scale(p, p, e0sb)
                    issue_scatter(p, p)
                    issue_gather(p + 2, p, ci)

                def pair(g, carry3):
                    for p in range(2):
                        b = 2 * g + p
                        wait_gather()
                        wait_scatter()
                        scale(b, p, e0sb)
                        issue_scatter(b, p)
                        issue_gather(b + 2, p, ci)
                    return carry3
                lax.fori_loop(1, nbp - 1, pair, 0)
                for p in range(2):
                    b = nb_sb - 2 + p
                    wait_gather()
                    wait_scatter()
                    scale(b, p, e0sb)
                    issue_scatter(b, p)
                wait_scatter()
                wait_scatter()
                return carry2
            lax.fori_loop(0, nsb, sb_body, 0)
            pltpu.sync_copy(dtab, dstage.at[s])
            plsc.subcore_barrier()
            for i in range(NS):
                off = i * 640
                sz = 640 if i < NS - 1 else Nn - 640 * (NS - 1)

                def dwork(off=off, sz=sz):
                    pltpu.sync_copy(dstage.at[0, pl.ds(off, sz)],
                                    dsum.at[pl.ds(0, sz)])

                    def dred(r, carry2):
                        pltpu.sync_copy(dstage.at[r, pl.ds(off, sz)],
                                        dstrip.at[pl.ds(0, sz)])

                        def dacc(v, carry3):
                            dsum[pl.ds(v * LN, LN)] = (
                                dsum[pl.ds(v * LN, LN)] +
                                dstrip[pl.ds(v * LN, LN)])
                            return carry3
                        lax.fori_loop(0, sz // LN, dacc, 0)
                        return carry2
                    lax.fori_loop(1, NS, dred, 0)
                    pltpu.sync_copy(dsum.at[pl.ds(0, sz)],
                                    den_hbm.at[ci, pl.ds(off, sz)])
                pl.when(s == i)(dwork)
            pltpu.sync_copy(acc.at[pl.ds(s * rpt, rpt)],
                            out_hbm.at[ci, pl.ds(s * rpt, rpt)])
            plsc.subcore_barrier()
            return carry
        lax.fori_loop(0, cpc, chunk_body, 0)

    return k


def _gat_tables(asrc, adst, n_chunks):
    Nn, H = asrc.shape
    hpc = H // n_chunks
    if n_chunks > 1:
        f = lambda a: (a.reshape(Nn, n_chunks, hpc)
                       .transpose(1, 0, 2).reshape(n_chunks, hpc * Nn))
        return f(asrc), f(adst)
    return asrc.reshape(Nn * H), adst.reshape(Nn * H)


def kernel(x, edge_index, W1, as1, ad1, b1, W2, as2, ad2, b2,
           W3, as3, ad3, b3, cW1, cb1, cW2, cb2):
    Nn, F = x.shape
    E = edge_index.shape[1]
    Ereal = E + Nn
    EP = ((Ereal + 165888 - 1) // 165888) * 165888
    loop = jnp.arange(Nn, dtype=I32)
    pad = jnp.zeros((EP - Ereal,), I32)
    src = jnp.concatenate([edge_index[0].astype(I32), loop, pad])
    dst = jnp.concatenate([edge_index[1].astype(I32), loop, pad])
    src12, dst12 = src.reshape(-1, 96), dst.reshape(-1, 96)
    src3, dst3 = src.reshape(-1, 64), dst.reshape(-1, 64)

    H, C = 8, 64
    HC = H * C
    S8 = (jnp.arange(HC)[:, None] // C == jnp.arange(H)[None, :]).astype(F32)
    S1 = jnp.ones((C, 1), F32)

    h4, asrc, adst = _tc_in(x, W1, as1.reshape(1, HC), ad1.reshape(1, HC),
                            S8, n_chunks=8)
    ts, td = _gat_tables(asrc, adst, 8)
    sc12 = _sc_edge_kernel(Nn, EP, Ereal, n_chunks=8, cpc=4, cw=64,
                           K=96, SB=5184, epw=EP // NS,
                           split_edges=False, tabw=Nn)
    acc1, den1 = sc12(h4.reshape(8 * Nn, 64), src12, dst12, ts, td)
    den1 = den1.T

    h4, asrc, adst = _tc_mid(acc1, den1, b1.reshape(1, HC), W2,
                             as2.reshape(1, HC), ad2.reshape(1, HC), S8,
                             n_chunks=8, cw=64, out_chunks=8)
    ts, td = _gat_tables(asrc, adst, 8)
    acc2, den2 = sc12(h4.reshape(8 * Nn, 64), src12, dst12, ts, td)
    den2 = den2.T

    h1, asrc, adst = _tc_mid(acc2, den2, b2.reshape(1, HC), W3,
                             as3.reshape(1, C), ad3.reshape(1, C), S1,
                             n_chunks=8, cw=64, out_chunks=1)
    ts, td = _gat_tables(asrc, adst, 1)
    sc3 = _sc_edge_kernel(Nn, EP, Ereal, n_chunks=1, cpc=1, cw=64,
                          K=64, SB=3456, epw=EP // (NC * NS),
                          split_edges=True, tabw=Nn)
    acc3, den3 = sc3(h1.reshape(Nn, 64), src3, dst3, ts, td)
    den3 = den3.T

    return _tc_final(acc3, den3, b3.reshape(1, C), cW1, cb1.reshape(1, -1),
                     cW2, cb2.reshape(1, -1), cw=64)


# speedup vs baseline: 1.4115x; 1.4115x over previous
import functools

import jax
import jax.numpy as jnp
from jax import lax
from jax.experimental import pallas as pl
from jax.experimental.pallas import tpu as pltpu
from jax.experimental.pallas import tpu_sc as plsc

F32 = jnp.float32
I32 = jnp.int32

NC = 2
NS = 16
LN = 16


def _attn_proj(a, asv, adv, S):
    asrc = jnp.dot(a * asv, S, preferred_element_type=F32)
    adst = jnp.dot(a * adv, S, preferred_element_type=F32)
    return asrc, adst


def _tc_in_body(x_ref, w_ref, asv_ref, adv_ref, S_ref,
                h4_ref, asrc_ref, adst_ref, *, n_chunks):
    a = jnp.dot(x_ref[...], w_ref[...], preferred_element_type=F32)
    cw = a.shape[1] // n_chunks
    for ci in range(n_chunks):
        h4_ref[ci] = a[:, ci * cw:(ci + 1) * cw]
    asrc_ref[...], adst_ref[...] = _attn_proj(a, asv_ref[...], adv_ref[...],
                                              S_ref[...])


def _tc_in(x, W, asv, adv, S, n_chunks, bn=1000):
    Nn, K = x.shape
    bn = bn if Nn % bn == 0 else Nn
    HC = W.shape[1]
    H = S.shape[1]
    cw = HC // n_chunks
    return pl.pallas_call(
        functools.partial(_tc_in_body, n_chunks=n_chunks),
        grid=(Nn // bn,),
        in_specs=[pl.BlockSpec((bn, K), lambda i: (i, 0)),
                  pl.BlockSpec((K, HC), lambda i: (0, 0)),
                  pl.BlockSpec((1, HC), lambda i: (0, 0)),
                  pl.BlockSpec((1, HC), lambda i: (0, 0)),
                  pl.BlockSpec((HC, H), lambda i: (0, 0))],
        out_specs=[pl.BlockSpec((n_chunks, bn, cw), lambda i: (0, i, 0)),
                   pl.BlockSpec((bn, H), lambda i: (i, 0)),
                   pl.BlockSpec((bn, H), lambda i: (i, 0))],
        out_shape=[jax.ShapeDtypeStruct((n_chunks, Nn, cw), F32),
                   jax.ShapeDtypeStruct((Nn, H), F32),
                   jax.ShapeDtypeStruct((Nn, H), F32)],
    )(x, W, asv, adv, S)


def _normalize_block(acc, den, n_chunks, cw):
    feats = [acc[ci] / den[:, ci:ci + 1] for ci in range(n_chunks)]
    return jnp.concatenate(feats, axis=1) if len(feats) > 1 else feats[0]


def _tc_mid_body(acc_ref, den_ref, b_ref, w_ref, asv_ref, adv_ref, S_ref,
                 h4_ref, asrc_ref, adst_ref, *, n_chunks, cw, out_chunks):
    feat = _normalize_block(acc_ref[...], den_ref[...], n_chunks, cw)
    xx = jnp.maximum(feat + b_ref[...], 0.0)
    a = jnp.dot(xx, w_ref[...], preferred_element_type=F32)
    ocw = a.shape[1] // out_chunks
    for ci in range(out_chunks):
        h4_ref[ci] = a[:, ci * ocw:(ci + 1) * ocw]
    asrc_ref[...], adst_ref[...] = _attn_proj(a, asv_ref[...], adv_ref[...],
                                              S_ref[...])


def _tc_mid(acc, den, b, W, asv, adv, S, n_chunks, cw, out_chunks, bn=1000):
    Nn = acc.shape[1]
    bn = bn if Nn % bn == 0 else Nn
    HCin = n_chunks * cw
    HCout = W.shape[1]
    H = S.shape[1]
    ocw = HCout // out_chunks
    return pl.pallas_call(
        functools.partial(_tc_mid_body, n_chunks=n_chunks, cw=cw,
                          out_chunks=out_chunks),
        grid=(Nn // bn,),
        in_specs=[pl.BlockSpec((n_chunks, bn, cw), lambda i: (0, i, 0)),
                  pl.BlockSpec((bn, n_chunks), lambda i: (i, 0)),
                  pl.BlockSpec((1, HCin), lambda i: (0, 0)),
                  pl.BlockSpec((HCin, HCout), lambda i: (0, 0)),
                  pl.BlockSpec((1, HCout), lambda i: (0, 0)),
                  pl.BlockSpec((1, HCout), lambda i: (0, 0)),
                  pl.BlockSpec((HCout, H), lambda i: (0, 0))],
        out_specs=[pl.BlockSpec((out_chunks, bn, ocw), lambda i: (0, i, 0)),
                   pl.BlockSpec((bn, H), lambda i: (i, 0)),
                   pl.BlockSpec((bn, H), lambda i: (i, 0))],
        out_shape=[jax.ShapeDtypeStruct((out_chunks, Nn, ocw), F32),
                   jax.ShapeDtypeStruct((Nn, H), F32),
                   jax.ShapeDtypeStruct((Nn, H), F32)],
    )(acc, den, b, W, asv, adv, S)


def _tc_final_body(acc_ref, den_ref, b3_ref, cW1_ref, cb1_ref, cW2_ref,
                   cb2_ref, out_ref, *, cw):
    p = acc_ref[0] + acc_ref[1]
    d = den_ref[:, 0:1] + den_ref[:, 1:2]
    feat = p / d
    h3 = feat + b3_ref[...]
    g = jnp.mean(h3, axis=0, keepdims=True)
    g1 = jnp.maximum(jnp.dot(g, cW1_ref[...],
                             preferred_element_type=F32) + cb1_ref[...], 0.0)
    logits = jnp.dot(g1, cW2_ref[...],
                     preferred_element_type=F32) + cb2_ref[...]
    m = jnp.max(logits, axis=1, keepdims=True)
    ls = m + jnp.log(jnp.sum(jnp.exp(logits - m), axis=1, keepdims=True))
    out_ref[...] = logits - ls


def _tc_final(acc3, den3, b3, cW1, cb1, cW2, cb2, cw):
    return pl.pallas_call(
        functools.partial(_tc_final_body, cw=cw),
        out_shape=jax.ShapeDtypeStruct((1, 2), F32),
    )(acc3, den3, b3, cW1, cb1, cW2, cb2)


def _sc_edge_kernel(Nn, EP, Ereal, n_chunks, cpc, cw, K, SB, epw,
                    split_edges, tabw):
    rpt = Nn // NS
    zr = 5
    nvec = K // LN
    nb_sb = SB // K
    nbp = nb_sb // 2
    nsb = epw // SB
    assert nb_sb % 2 == 0 and epw % SB == 0 and SB % K == 0
    mesh = plsc.VectorSubcoreMesh(core_axis_name="c", subcore_axis_name="s",
                                  num_cores=NC, num_subcores=NS)
    out_rows = NC * cpc

    @functools.partial(
        pl.kernel, mesh=mesh,
        compiler_params=pltpu.CompilerParams(use_tc_tiling_on_sc=False,
                                             needs_layout_passes=False),
        out_type=[jax.ShapeDtypeStruct((out_rows, Nn, cw), F32),
                  jax.ShapeDtypeStruct((out_rows, Nn), F32)],
        scratch_types=[
            pltpu.VMEM_SHARED((Nn, cw), F32),
            pltpu.VMEM_SHARED((NS, Nn), F32),
            pltpu.VMEM((tabw,), F32),
            pltpu.VMEM((tabw,), F32),
            pltpu.VMEM((Nn,), F32),
            pltpu.VMEM((640,), F32),
            pltpu.VMEM((640,), F32),
            pltpu.VMEM((nb_sb, K), I32),
            pltpu.VMEM((nb_sb, K), I32),
            pltpu.VMEM((K,), I32),
            pltpu.VMEM((K,), I32),
            pltpu.VMEM((2, K, cw), F32),
            pltpu.VMEM((2, K, cw), F32),
            pltpu.VMEM((zr, cw), F32),
            pltpu.SemaphoreType.DMA,
            pltpu.SemaphoreType.DMA,
        ],
    )
    def k(h_hbm, src_hbm, dst_hbm, as_hbm, ad_hbm, out_hbm, den_hbm,
          acc, dstage, tabs, tabd, dtab, dstrip, dsum,
          sidx, didx, gidx0, gidx1, rows, sbuf,
          zbuf, sem_g, sem_s):
        c = lax.axis_index("c")
        s = lax.axis_index("s")
        zero16 = jnp.zeros((LN,), F32)
        for r in range(zr):
            for q in range(cw // LN):
                zbuf[r, pl.ds(q * LN, LN)] = zero16
        if split_edges:
            ebase = (c * NS + s) * epw
        else:
            ebase = s * epw
        brow0 = ebase // K
        iota16 = lax.iota(I32, LN)
        gixb = (gidx0, gidx1)

        def issue_gather(b, p, ci):
            if n_chunks > 1:
                def gix(t, carry):
                    sv = sidx[b, pl.ds(t * LN, LN)]
                    gixb[p][pl.ds(t * LN, LN)] = sv + ci * Nn
                    return carry
                lax.fori_loop(0, nvec, gix, 0)
                pltpu.async_copy(h_hbm.at[gixb[p]], rows.at[p], sem_g)
            else:
                pltpu.async_copy(h_hbm.at[sidx.at[b]], rows.at[p], sem_g)

        def wait_gather():
            pltpu.make_async_copy(h_hbm.at[pl.ds(0, K)], rows.at[0],
                                  sem_g).wait()

        def issue_scatter(b, p):
            pltpu.async_copy(sbuf.at[p], acc.at[didx.at[b]], sem_s,
                             add=True)

        def wait_scatter():
            pltpu.make_async_copy(sbuf.at[0], acc.at[pl.ds(0, K)],
                                  sem_s).wait()

        def scale(b, p, e0sb):
            def scale16(t, carry):
                sv = sidx[b, pl.ds(t * LN, LN)]
                dv = didx[b, pl.ds(t * LN, LN)]
                valid = (e0sb + b * K + t * LN + iota16) < Ereal
                gs = plsc.load_gather(tabs, [sv])
                gd = plsc.load_gather(tabd, [dv])
                al = gs + gd
                al = jnp.maximum(al, 0.2 * al)
                wv = jnp.where(valid, jnp.exp(al), 0.0)
                plsc.addupdate_scatter(dtab, [dv], wv)
                for lane in range(LN):
                    kk = t * LN + lane
                    a0 = wv[lane]
                    v0 = jnp.full((LN,), a0, F32)
                    for q in range(cw // LN):
                        sbuf[p, kk, pl.ds(q * LN, LN)] = (
                            rows[p, kk, pl.ds(q * LN, LN)] * v0)
                return carry

            def scale16_pl(t):
                scale16(t, 0)
            plsc.parallel_loop(0, nvec)(scale16_pl)

        def chunk_body(j, carry):
            ci = c * cpc + j
            for z in range(rpt // zr):
                pltpu.sync_copy(zbuf, acc.at[pl.ds(s * rpt + z * zr, zr)])

            def dzero(r, carry2):
                dtab[pl.ds(r * LN, LN)] = zero16
                return carry2
            lax.fori_loop(0, Nn // LN, dzero, 0)
            plsc.subcore_barrier()
            if n_chunks > 1:
                pltpu.sync_copy(as_hbm.at[ci], tabs)
                pltpu.sync_copy(ad_hbm.at[ci], tabd)
            else:
                pltpu.sync_copy(as_hbm, tabs)
                pltpu.sync_copy(ad_hbm, tabd)

            def sb_body(q, carry2):
                row0 = brow0 + q * nb_sb
                pltpu.sync_copy(src_hbm.at[pl.ds(row0, nb_sb)], sidx)
                pltpu.sync_copy(dst_hbm.at[pl.ds(row0, nb_sb)], didx)
                e0sb = ebase + q * SB
                issue_gather(0, 0, ci)
                issue_gather(1, 1, ci)
                for p in range(2):
                    wait_gather()
                    scale(p, p, e0sb)
                    issue_scatter(p, p)
                    issue_gather(p + 2, p, ci)

                def pair(g, carry3):
                    for p in range(2):
                        b = 2 * g + p
                        wait_gather()
                        wait_scatter()
                        scale(b, p, e0sb)
                        issue_scatter(b, p)
                        issue_gather(b + 2, p, ci)
                    return carry3
                lax.fori_loop(1, nbp - 1, pair, 0)
                for p in range(2):
                    b = nb_sb - 2 + p
                    wait_gather()
                    wait_scatter()
                    scale(b, p, e0sb)
                    issue_scatter(b, p)
                wait_scatter()
                wait_scatter()
                return carry2
            lax.fori_loop(0, nsb, sb_body, 0)
            pltpu.sync_copy(dtab, dstage.at[s])
            plsc.subcore_barrier()
            for i in range(NS):
                off = i * 640
                sz = 640 if i < NS - 1 else Nn - 640 * (NS - 1)

                def dwork(off=off, sz=sz):
                    pltpu.sync_copy(dstage.at[0, pl.ds(off, sz)],
                                    dsum.at[pl.ds(0, sz)])

                    def dred(r, carry2):
                        pltpu.sync_copy(dstage.at[r, pl.ds(off, sz)],
                                        dstrip.at[pl.ds(0, sz)])

                        def dacc(v, carry3):
                            dsum[pl.ds(v * LN, LN)] = (
                                dsum[pl.ds(v * LN, LN)] +
                                dstrip[pl.ds(v * LN, LN)])
                            return carry3
                        lax.fori_loop(0, sz // LN, dacc, 0)
                        return carry2
                    lax.fori_loop(1, NS, dred, 0)
                    pltpu.sync_copy(dsum.at[pl.ds(0, sz)],
                                    den_hbm.at[ci, pl.ds(off, sz)])
                pl.when(s == i)(dwork)
            pltpu.sync_copy(acc.at[pl.ds(s * rpt, rpt)],
                            out_hbm.at[ci, pl.ds(s * rpt, rpt)])
            plsc.subcore_barrier()
            return carry
        lax.fori_loop(0, cpc, chunk_body, 0)

    return k


def _gat_tables(asrc, adst, n_chunks):
    Nn, H = asrc.shape
    hpc = H // n_chunks
    if n_chunks > 1:
        f = lambda a: (a.reshape(Nn, n_chunks, hpc)
                       .transpose(1, 0, 2).reshape(n_chunks, hpc * Nn))
        return f(asrc), f(adst)
    return asrc.reshape(Nn * H), adst.reshape(Nn * H)


def kernel(x, edge_index, W1, as1, ad1, b1, W2, as2, ad2, b2,
           W3, as3, ad3, b3, cW1, cb1, cW2, cb2):
    Nn, F = x.shape
    E = edge_index.shape[1]
    Ereal = E + Nn
    EP = ((Ereal + 165888 - 1) // 165888) * 165888
    loop = jnp.arange(Nn, dtype=I32)
    pad = jnp.zeros((EP - Ereal,), I32)
    src = jnp.concatenate([edge_index[0].astype(I32), loop, pad])
    dst = jnp.concatenate([edge_index[1].astype(I32), loop, pad])
    src12, dst12 = src.reshape(-1, 96), dst.reshape(-1, 96)
    src3, dst3 = src.reshape(-1, 64), dst.reshape(-1, 64)

    H, C = 8, 64
    HC = H * C
    S8 = (jnp.arange(HC)[:, None] // C == jnp.arange(H)[None, :]).astype(F32)
    S1 = jnp.ones((C, 1), F32)

    h4, asrc, adst = _tc_in(x, W1, as1.reshape(1, HC), ad1.reshape(1, HC),
                            S8, n_chunks=8)
    ts, td = _gat_tables(asrc, adst, 8)
    sc12 = _sc_edge_kernel(Nn, EP, Ereal, n_chunks=8, cpc=4, cw=64,
                           K=96, SB=5184, epw=EP // NS,
                           split_edges=False, tabw=Nn)
    acc1, den1 = sc12(h4.reshape(8 * Nn, 64), src12, dst12, ts, td)
    den1 = den1.T

    h4, asrc, adst = _tc_mid(acc1, den1, b1.reshape(1, HC), W2,
                             as2.reshape(1, HC), ad2.reshape(1, HC), S8,
                             n_chunks=8, cw=64, out_chunks=8)
    ts, td = _gat_tables(asrc, adst, 8)
    acc2, den2 = sc12(h4.reshape(8 * Nn, 64), src12, dst12, ts, td)
    den2 = den2.T

    h1, asrc, adst = _tc_mid(acc2, den2, b2.reshape(1, HC), W3,
                             as3.reshape(1, C), ad3.reshape(1, C), S1,
                             n_chunks=8, cw=64, out_chunks=1)
    ts, td = _gat_tables(asrc, adst, 1)
    sc3 = _sc_edge_kernel(Nn, EP, Ereal, n_chunks=1, cpc=1, cw=64,
                          K=64, SB=3456, epw=EP // (NC * NS),
                          split_edges=True, tabw=Nn)
    acc3, den3 = sc3(h1.reshape(Nn, 64), src3, dst3, ts, td)
    den3 = den3.T

    return _tc_final(acc3, den3, b3.reshape(1, C), cW1, cb1.reshape(1, -1),
                     cW2, cb2.reshape(1, -1), cw=64)


# speedup vs baseline: 2.0277x; 1.4366x over previous
import functools

import jax
import jax.numpy as jnp
from jax import lax
from jax.experimental import pallas as pl
from jax.experimental.pallas import tpu as pltpu
from jax.experimental.pallas import tpu_sc as plsc

F32 = jnp.float32
I32 = jnp.int32

NC = 2
NS = 16
LN = 16


def _attn_proj(a, asv, adv, S):
    asrc = jnp.dot(a * asv, S, preferred_element_type=F32)
    adst = jnp.dot(a * adv, S, preferred_element_type=F32)
    return asrc, adst


def _tc_in_body(x_ref, w_ref, asv_ref, adv_ref, S_ref,
                h4_ref, asrc_ref, adst_ref, *, n_chunks):
    a = jnp.dot(x_ref[...], w_ref[...], preferred_element_type=F32)
    cw = a.shape[1] // n_chunks
    for ci in range(n_chunks):
        h4_ref[ci] = a[:, ci * cw:(ci + 1) * cw]
    asrc_ref[...], adst_ref[...] = _attn_proj(a, asv_ref[...], adv_ref[...],
                                              S_ref[...])


def _tc_in(x, W, asv, adv, S, n_chunks, bn=1000):
    Nn, K = x.shape
    bn = bn if Nn % bn == 0 else Nn
    HC = W.shape[1]
    H = S.shape[1]
    cw = HC // n_chunks
    return pl.pallas_call(
        functools.partial(_tc_in_body, n_chunks=n_chunks),
        grid=(Nn // bn,),
        in_specs=[pl.BlockSpec((bn, K), lambda i: (i, 0)),
                  pl.BlockSpec((K, HC), lambda i: (0, 0)),
                  pl.BlockSpec((1, HC), lambda i: (0, 0)),
                  pl.BlockSpec((1, HC), lambda i: (0, 0)),
                  pl.BlockSpec((HC, H), lambda i: (0, 0))],
        out_specs=[pl.BlockSpec((n_chunks, bn, cw), lambda i: (0, i, 0)),
                   pl.BlockSpec((bn, H), lambda i: (i, 0)),
                   pl.BlockSpec((bn, H), lambda i: (i, 0))],
        out_shape=[jax.ShapeDtypeStruct((n_chunks, Nn, cw), F32),
                   jax.ShapeDtypeStruct((Nn, H), F32),
                   jax.ShapeDtypeStruct((Nn, H), F32)],
    )(x, W, asv, adv, S)


def _normalize_block(acc, den, n_chunks, cw):
    feats = [acc[ci] / den[:, ci:ci + 1] for ci in range(n_chunks)]
    return jnp.concatenate(feats, axis=1) if len(feats) > 1 else feats[0]


def _tc_mid_body(acc_ref, den_ref, b_ref, w_ref, asv_ref, adv_ref, S_ref,
                 h4_ref, asrc_ref, adst_ref, *, n_chunks, cw, out_chunks):
    feat = _normalize_block(acc_ref[...], den_ref[...], n_chunks, cw)
    xx = jnp.maximum(feat + b_ref[...], 0.0)
    a = jnp.dot(xx, w_ref[...], preferred_element_type=F32)
    ocw = a.shape[1] // out_chunks
    for ci in range(out_chunks):
        h4_ref[ci] = a[:, ci * ocw:(ci + 1) * ocw]
    asrc_ref[...], adst_ref[...] = _attn_proj(a, asv_ref[...], adv_ref[...],
                                              S_ref[...])


def _tc_mid(acc, den, b, W, asv, adv, S, n_chunks, cw, out_chunks, bn=1000):
    Nn = acc.shape[1]
    bn = bn if Nn % bn == 0 else Nn
    HCin = n_chunks * cw
    HCout = W.shape[1]
    H = S.shape[1]
    ocw = HCout // out_chunks
    return pl.pallas_call(
        functools.partial(_tc_mid_body, n_chunks=n_chunks, cw=cw,
                          out_chunks=out_chunks),
        grid=(Nn // bn,),
        in_specs=[pl.BlockSpec((n_chunks, bn, cw), lambda i: (0, i, 0)),
                  pl.BlockSpec((bn, n_chunks), lambda i: (i, 0)),
                  pl.BlockSpec((1, HCin), lambda i: (0, 0)),
                  pl.BlockSpec((HCin, HCout), lambda i: (0, 0)),
                  pl.BlockSpec((1, HCout), lambda i: (0, 0)),
                  pl.BlockSpec((1, HCout), lambda i: (0, 0)),
                  pl.BlockSpec((HCout, H), lambda i: (0, 0))],
        out_specs=[pl.BlockSpec((out_chunks, bn, ocw), lambda i: (0, i, 0)),
                   pl.BlockSpec((bn, H), lambda i: (i, 0)),
                   pl.BlockSpec((bn, H), lambda i: (i, 0))],
        out_shape=[jax.ShapeDtypeStruct((out_chunks, Nn, ocw), F32),
                   jax.ShapeDtypeStruct((Nn, H), F32),
                   jax.ShapeDtypeStruct((Nn, H), F32)],
    )(acc, den, b, W, asv, adv, S)


def _tc_final_body(acc_ref, den_ref, b3_ref, cW1_ref, cb1_ref, cW2_ref,
                   cb2_ref, out_ref, *, cw):
    p = acc_ref[0] + acc_ref[1]
    d = den_ref[:, 0:1] + den_ref[:, 1:2]
    feat = p / d
    h3 = feat + b3_ref[...]
    g = jnp.mean(h3, axis=0, keepdims=True)
    g1 = jnp.maximum(jnp.dot(g, cW1_ref[...],
                             preferred_element_type=F32) + cb1_ref[...], 0.0)
    logits = jnp.dot(g1, cW2_ref[...],
                     preferred_element_type=F32) + cb2_ref[...]
    m = jnp.max(logits, axis=1, keepdims=True)
    ls = m + jnp.log(jnp.sum(jnp.exp(logits - m), axis=1, keepdims=True))
    out_ref[...] = logits - ls


def _tc_final(acc3, den3, b3, cW1, cb1, cW2, cb2, cw):
    return pl.pallas_call(
        functools.partial(_tc_final_body, cw=cw),
        out_shape=jax.ShapeDtypeStruct((1, 2), F32),
    )(acc3, den3, b3, cW1, cb1, cW2, cb2)


def _sc_edge_kernel(Nn, EP, Ereal, n_chunks, cpc, cw, K, SB, epw,
                    split_edges, tabw):
    rpt = Nn // NS
    zr = 5
    nvec = K // LN
    nb_sb = SB // K
    nbp = nb_sb // 2
    nsb = epw // SB
    assert nb_sb % 2 == 0 and epw % SB == 0 and SB % K == 0
    mesh = plsc.VectorSubcoreMesh(core_axis_name="c", subcore_axis_name="s",
                                  num_cores=NC, num_subcores=NS)
    out_rows = NC * cpc

    @functools.partial(
        pl.kernel, mesh=mesh,
        compiler_params=pltpu.CompilerParams(use_tc_tiling_on_sc=False,
                                             needs_layout_passes=False),
        out_type=[jax.ShapeDtypeStruct((out_rows, Nn, cw), F32),
                  jax.ShapeDtypeStruct((out_rows, Nn), F32)],
        scratch_types=[
            pltpu.VMEM_SHARED((Nn, cw), F32),
            pltpu.VMEM_SHARED((NS, Nn), F32),
            pltpu.VMEM((tabw,), F32),
            pltpu.VMEM((tabw,), F32),
            pltpu.VMEM((Nn,), F32),
            pltpu.VMEM((640,), F32),
            pltpu.VMEM((640,), F32),
            pltpu.VMEM((nb_sb, K), I32),
            pltpu.VMEM((nb_sb, K), I32),
            pltpu.VMEM((K,), I32),
            pltpu.VMEM((K,), I32),
            pltpu.VMEM((2, K, cw), F32),
            pltpu.VMEM((2, K, cw), F32),
            pltpu.VMEM((zr, cw), F32),
            pltpu.SemaphoreType.DMA,
            pltpu.SemaphoreType.DMA,
        ],
    )
    def k(h_hbm, src_hbm, dst_hbm, as_hbm, ad_hbm, out_hbm, den_hbm,
          acc, dstage, tabs, tabd, dtab, dstrip, dsum,
          sidx, didx, gidx0, gidx1, rows, sbuf,
          zbuf, sem_g, sem_s):
        c = lax.axis_index("c")
        s = lax.axis_index("s")
        zero16 = jnp.zeros((LN,), F32)
        for r in range(zr):
            for q in range(cw // LN):
                zbuf[r, pl.ds(q * LN, LN)] = zero16
        if split_edges:
            ebase = (c * NS + s) * epw
        else:
            ebase = s * epw
        brow0 = ebase // K
        iota16 = lax.iota(I32, LN)
        gixb = (gidx0, gidx1)

        def issue_gather(b, p, ci):
            if n_chunks > 1:
                def gix(t):
                    sv = sidx[b, pl.ds(t * LN, LN)]
                    gixb[p][pl.ds(t * LN, LN)] = sv + ci * Nn
                plsc.parallel_loop(0, nvec, unroll=2)(gix)
                pltpu.async_copy(h_hbm.at[gixb[p]], rows.at[p], sem_g)
            else:
                pltpu.async_copy(h_hbm.at[sidx.at[b]], rows.at[p], sem_g)

        def wait_gather():
            pltpu.make_async_copy(h_hbm.at[pl.ds(0, K)], rows.at[0],
                                  sem_g).wait()

        def issue_scatter(b, p):
            pltpu.async_copy(sbuf.at[p], acc.at[didx.at[b]], sem_s,
                             add=True)

        def wait_scatter():
            pltpu.make_async_copy(sbuf.at[0], acc.at[pl.ds(0, K)],
                                  sem_s).wait()

        def scale(b, p, e0sb):
            def scale16(t, carry):
                sv = sidx[b, pl.ds(t * LN, LN)]
                dv = didx[b, pl.ds(t * LN, LN)]
                valid = (e0sb + b * K + t * LN + iota16) < Ereal
                gs = plsc.load_gather(tabs, [sv])
                gd = plsc.load_gather(tabd, [dv])
                al = gs + gd
                al = jnp.maximum(al, 0.2 * al)
                wv = jnp.where(valid, jnp.exp(al), 0.0)
                plsc.addupdate_scatter(dtab, [dv], wv)
                for lane in range(LN):
                    kk = t * LN + lane
                    a0 = wv[lane]
                    v0 = jnp.full((LN,), a0, F32)
                    for q in range(cw // LN):
                        sbuf[p, kk, pl.ds(q * LN, LN)] = (
                            rows[p, kk, pl.ds(q * LN, LN)] * v0)
                return carry

            def scale16_pl(t):
                scale16(t, 0)
            plsc.parallel_loop(0, nvec, unroll=2)(scale16_pl)

        def chunk_body(j, carry):
            ci = c * cpc + j
            for z in range(rpt // zr):
                pltpu.sync_copy(zbuf, acc.at[pl.ds(s * rpt + z * zr, zr)])

            def dzero(r, carry2):
                dtab[pl.ds(r * LN, LN)] = zero16
                return carry2
            lax.fori_loop(0, Nn // LN, dzero, 0)
            plsc.subcore_barrier()
            if n_chunks > 1:
                pltpu.sync_copy(as_hbm.at[ci], tabs)
                pltpu.sync_copy(ad_hbm.at[ci], tabd)
            else:
                pltpu.sync_copy(as_hbm, tabs)
                pltpu.sync_copy(ad_hbm, tabd)

            def sb_body(q, carry2):
                row0 = brow0 + q * nb_sb
                pltpu.sync_copy(src_hbm.at[pl.ds(row0, nb_sb)], sidx)
                pltpu.sync_copy(dst_hbm.at[pl.ds(row0, nb_sb)], didx)
                e0sb = ebase + q * SB
                issue_gather(0, 0, ci)
                issue_gather(1, 1, ci)
                for p in range(2):
                    wait_gather()
                    scale(p, p, e0sb)
                    issue_scatter(p, p)
                    issue_gather(p + 2, p, ci)

                def pair(g, carry3):
                    for p in range(2):
                        b = 2 * g + p
                        wait_gather()
                        wait_scatter()
                        scale(b, p, e0sb)
                        issue_scatter(b, p)
                        issue_gather(b + 2, p, ci)
                    return carry3
                lax.fori_loop(1, nbp - 1, pair, 0)
                for p in range(2):
                    b = nb_sb - 2 + p
                    wait_gather()
                    wait_scatter()
                    scale(b, p, e0sb)
                    issue_scatter(b, p)
                wait_scatter()
                wait_scatter()
                return carry2
            lax.fori_loop(0, nsb, sb_body, 0)
            pltpu.sync_copy(dtab, dstage.at[s])
            plsc.subcore_barrier()
            for i in range(NS):
                off = i * 640
                sz = 640 if i < NS - 1 else Nn - 640 * (NS - 1)

                def dwork(off=off, sz=sz):
                    pltpu.sync_copy(dstage.at[0, pl.ds(off, sz)],
                                    dsum.at[pl.ds(0, sz)])

                    def dred(r, carry2):
                        pltpu.sync_copy(dstage.at[r, pl.ds(off, sz)],
                                        dstrip.at[pl.ds(0, sz)])

                        def dacc(v, carry3):
                            dsum[pl.ds(v * LN, LN)] = (
                                dsum[pl.ds(v * LN, LN)] +
                                dstrip[pl.ds(v * LN, LN)])
                            return carry3
                        lax.fori_loop(0, sz // LN, dacc, 0)
                        return carry2
                    lax.fori_loop(1, NS, dred, 0)
                    pltpu.sync_copy(dsum.at[pl.ds(0, sz)],
                                    den_hbm.at[ci, pl.ds(off, sz)])
                pl.when(s == i)(dwork)
            pltpu.sync_copy(acc.at[pl.ds(s * rpt, rpt)],
                            out_hbm.at[ci, pl.ds(s * rpt, rpt)])
            plsc.subcore_barrier()
            return carry
        lax.fori_loop(0, cpc, chunk_body, 0)

    return k


def _gat_tables(asrc, adst, n_chunks):
    Nn, H = asrc.shape
    hpc = H // n_chunks
    if n_chunks > 1:
        f = lambda a: (a.reshape(Nn, n_chunks, hpc)
                       .transpose(1, 0, 2).reshape(n_chunks, hpc * Nn))
        return f(asrc), f(adst)
    return asrc.reshape(Nn * H), adst.reshape(Nn * H)


def kernel(x, edge_index, W1, as1, ad1, b1, W2, as2, ad2, b2,
           W3, as3, ad3, b3, cW1, cb1, cW2, cb2):
    Nn, F = x.shape
    E = edge_index.shape[1]
    Ereal = E + Nn
    EP = ((Ereal + 165888 - 1) // 165888) * 165888
    loop = jnp.arange(Nn, dtype=I32)
    pad = jnp.zeros((EP - Ereal,), I32)
    src = jnp.concatenate([edge_index[0].astype(I32), loop, pad])
    dst = jnp.concatenate([edge_index[1].astype(I32), loop, pad])
    src12, dst12 = src.reshape(-1, 96), dst.reshape(-1, 96)
    src3, dst3 = src.reshape(-1, 64), dst.reshape(-1, 64)

    H, C = 8, 64
    HC = H * C
    S8 = (jnp.arange(HC)[:, None] // C == jnp.arange(H)[None, :]).astype(F32)
    S1 = jnp.ones((C, 1), F32)

    h4, asrc, adst = _tc_in(x, W1, as1.reshape(1, HC), ad1.reshape(1, HC),
                            S8, n_chunks=8)
    ts, td = _gat_tables(asrc, adst, 8)
    sc12 = _sc_edge_kernel(Nn, EP, Ereal, n_chunks=8, cpc=4, cw=64,
                           K=96, SB=5184, epw=EP // NS,
                           split_edges=False, tabw=Nn)
    acc1, den1 = sc12(h4.reshape(8 * Nn, 64), src12, dst12, ts, td)
    den1 = den1.T

    h4, asrc, adst = _tc_mid(acc1, den1, b1.reshape(1, HC), W2,
                             as2.reshape(1, HC), ad2.reshape(1, HC), S8,
                             n_chunks=8, cw=64, out_chunks=8)
    ts, td = _gat_tables(asrc, adst, 8)
    acc2, den2 = sc12(h4.reshape(8 * Nn, 64), src12, dst12, ts, td)
    den2 = den2.T

    h1, asrc, adst = _tc_mid(acc2, den2, b2.reshape(1, HC), W3,
                             as3.reshape(1, C), ad3.reshape(1, C), S1,
                             n_chunks=8, cw=64, out_chunks=1)
    ts, td = _gat_tables(asrc, adst, 1)
    sc3 = _sc_edge_kernel(Nn, EP, Ereal, n_chunks=1, cpc=1, cw=64,
                          K=64, SB=3456, epw=EP // (NC * NS),
                          split_edges=True, tabw=Nn)
    acc3, den3 = sc3(h1.reshape(Nn, 64), src3, dst3, ts, td)
    den3 = den3.T

    return _tc_final(acc3, den3, b3.reshape(1, C), cW1, cb1.reshape(1, -1),
                     cW2, cb2.reshape(1, -1), cw=64)


# speedup vs baseline: 2.0304x; 1.0013x over previous
import functools

import jax
import jax.numpy as jnp
from jax import lax
from jax.experimental import pallas as pl
from jax.experimental.pallas import tpu as pltpu
from jax.experimental.pallas import tpu_sc as plsc

F32 = jnp.float32
I32 = jnp.int32

NC = 2
NS = 16
LN = 16


def _attn_proj(a, asv, adv, S):
    asrc = jnp.dot(a * asv, S, preferred_element_type=F32)
    adst = jnp.dot(a * adv, S, preferred_element_type=F32)
    return asrc, adst


def _tc_in_body(x_ref, w_ref, asv_ref, adv_ref, S_ref,
                h4_ref, asrc_ref, adst_ref, *, n_chunks):
    a = jnp.dot(x_ref[...], w_ref[...], preferred_element_type=F32)
    cw = a.shape[1] // n_chunks
    for ci in range(n_chunks):
        h4_ref[ci] = a[:, ci * cw:(ci + 1) * cw]
    asrc_ref[...], adst_ref[...] = _attn_proj(a, asv_ref[...], adv_ref[...],
                                              S_ref[...])


def _tc_in(x, W, asv, adv, S, n_chunks, bn=1000):
    Nn, K = x.shape
    bn = bn if Nn % bn == 0 else Nn
    HC = W.shape[1]
    H = S.shape[1]
    cw = HC // n_chunks
    return pl.pallas_call(
        functools.partial(_tc_in_body, n_chunks=n_chunks),
        grid=(Nn // bn,),
        in_specs=[pl.BlockSpec((bn, K), lambda i: (i, 0)),
                  pl.BlockSpec((K, HC), lambda i: (0, 0)),
                  pl.BlockSpec((1, HC), lambda i: (0, 0)),
                  pl.BlockSpec((1, HC), lambda i: (0, 0)),
                  pl.BlockSpec((HC, H), lambda i: (0, 0))],
        out_specs=[pl.BlockSpec((n_chunks, bn, cw), lambda i: (0, i, 0)),
                   pl.BlockSpec((bn, H), lambda i: (i, 0)),
                   pl.BlockSpec((bn, H), lambda i: (i, 0))],
        out_shape=[jax.ShapeDtypeStruct((n_chunks, Nn, cw), F32),
                   jax.ShapeDtypeStruct((Nn, H), F32),
                   jax.ShapeDtypeStruct((Nn, H), F32)],
    )(x, W, asv, adv, S)


def _normalize_block(acc, den, n_chunks, cw):
    feats = [acc[ci] / den[:, ci:ci + 1] for ci in range(n_chunks)]
    return jnp.concatenate(feats, axis=1) if len(feats) > 1 else feats[0]


def _tc_mid_body(acc_ref, den_ref, b_ref, w_ref, asv_ref, adv_ref, S_ref,
                 h4_ref, asrc_ref, adst_ref, *, n_chunks, cw, out_chunks):
    feat = _normalize_block(acc_ref[...], den_ref[...], n_chunks, cw)
    xx = jnp.maximum(feat + b_ref[...], 0.0)
    a = jnp.dot(xx, w_ref[...], preferred_element_type=F32)
    ocw = a.shape[1] // out_chunks
    for ci in range(out_chunks):
        h4_ref[ci] = a[:, ci * ocw:(ci + 1) * ocw]
    asrc_ref[...], adst_ref[...] = _attn_proj(a, asv_ref[...], adv_ref[...],
                                              S_ref[...])


def _tc_mid(acc, den, b, W, asv, adv, S, n_chunks, cw, out_chunks, bn=1000):
    Nn = acc.shape[1]
    bn = bn if Nn % bn == 0 else Nn
    HCin = n_chunks * cw
    HCout = W.shape[1]
    H = S.shape[1]
    ocw = HCout // out_chunks
    return pl.pallas_call(
        functools.partial(_tc_mid_body, n_chunks=n_chunks, cw=cw,
                          out_chunks=out_chunks),
        grid=(Nn // bn,),
        in_specs=[pl.BlockSpec((n_chunks, bn, cw), lambda i: (0, i, 0)),
                  pl.BlockSpec((bn, n_chunks), lambda i: (i, 0)),
                  pl.BlockSpec((1, HCin), lambda i: (0, 0)),
                  pl.BlockSpec((HCin, HCout), lambda i: (0, 0)),
                  pl.BlockSpec((1, HCout), lambda i: (0, 0)),
                  pl.BlockSpec((1, HCout), lambda i: (0, 0)),
                  pl.BlockSpec((HCout, H), lambda i: (0, 0))],
        out_specs=[pl.BlockSpec((out_chunks, bn, ocw), lambda i: (0, i, 0)),
                   pl.BlockSpec((bn, H), lambda i: (i, 0)),
                   pl.BlockSpec((bn, H), lambda i: (i, 0))],
        out_shape=[jax.ShapeDtypeStruct((out_chunks, Nn, ocw), F32),
                   jax.ShapeDtypeStruct((Nn, H), F32),
                   jax.ShapeDtypeStruct((Nn, H), F32)],
    )(acc, den, b, W, asv, adv, S)


def _tc_final_body(acc_ref, den_ref, b3_ref, cW1_ref, cb1_ref, cW2_ref,
                   cb2_ref, out_ref, *, cw):
    p = acc_ref[0] + acc_ref[1]
    d = den_ref[:, 0:1] + den_ref[:, 1:2]
    feat = p / d
    h3 = feat + b3_ref[...]
    g = jnp.mean(h3, axis=0, keepdims=True)
    g1 = jnp.maximum(jnp.dot(g, cW1_ref[...],
                             preferred_element_type=F32) + cb1_ref[...], 0.0)
    logits = jnp.dot(g1, cW2_ref[...],
                     preferred_element_type=F32) + cb2_ref[...]
    m = jnp.max(logits, axis=1, keepdims=True)
    ls = m + jnp.log(jnp.sum(jnp.exp(logits - m), axis=1, keepdims=True))
    out_ref[...] = logits - ls


def _tc_final(acc3, den3, b3, cW1, cb1, cW2, cb2, cw):
    return pl.pallas_call(
        functools.partial(_tc_final_body, cw=cw),
        out_shape=jax.ShapeDtypeStruct((1, 2), F32),
    )(acc3, den3, b3, cW1, cb1, cW2, cb2)


def _sc_edge_kernel(Nn, EP, Ereal, n_chunks, cpc, cw, K, SB, epw,
                    split_edges, tabw):
    rpt = Nn // NS
    zr = 5
    nvec = K // LN
    nb_sb = SB // K
    nbp = nb_sb // 2
    nsb = epw // SB
    assert nb_sb % 2 == 0 and epw % SB == 0 and SB % K == 0
    mesh = plsc.VectorSubcoreMesh(core_axis_name="c", subcore_axis_name="s",
                                  num_cores=NC, num_subcores=NS)
    out_rows = NC * cpc

    @functools.partial(
        pl.kernel, mesh=mesh,
        compiler_params=pltpu.CompilerParams(use_tc_tiling_on_sc=False,
                                             needs_layout_passes=False),
        out_type=[jax.ShapeDtypeStruct((out_rows, Nn, cw), F32),
                  jax.ShapeDtypeStruct((out_rows, Nn), F32)],
        scratch_types=[
            pltpu.VMEM_SHARED((Nn, cw), F32),
            pltpu.VMEM_SHARED((NS, Nn), F32),
            pltpu.VMEM((tabw,), F32),
            pltpu.VMEM((tabw,), F32),
            pltpu.VMEM((Nn,), F32),
            pltpu.VMEM((640,), F32),
            pltpu.VMEM((640,), F32),
            pltpu.VMEM((nb_sb, K), I32),
            pltpu.VMEM((nb_sb, K), I32),
            pltpu.VMEM((K,), I32),
            pltpu.VMEM((K,), I32),
            pltpu.VMEM((2, K, cw), F32),
            pltpu.VMEM((2, K, cw), F32),
            pltpu.VMEM((zr, cw), F32),
            pltpu.SemaphoreType.DMA,
            pltpu.SemaphoreType.DMA,
        ],
    )
    def k(h_hbm, src_hbm, dst_hbm, as_hbm, ad_hbm, out_hbm, den_hbm,
          acc, dstage, tabs, tabd, dtab, dstrip, dsum,
          sidx, didx, gidx0, gidx1, rows, sbuf,
          zbuf, sem_g, sem_s):
        c = lax.axis_index("c")
        s = lax.axis_index("s")
        zero16 = jnp.zeros((LN,), F32)
        for r in range(zr):
            for q in range(cw // LN):
                zbuf[r, pl.ds(q * LN, LN)] = zero16
        if split_edges:
            ebase = (c * NS + s) * epw
        else:
            ebase = s * epw
        brow0 = ebase // K
        iota16 = lax.iota(I32, LN)
        gixb = (gidx0, gidx1)

        def issue_gather(b, p, ci):
            if n_chunks > 1:
                def gix(t):
                    sv = sidx[b, pl.ds(t * LN, LN)]
                    gixb[p][pl.ds(t * LN, LN)] = sv + ci * Nn
                plsc.parallel_loop(0, nvec, unroll=2)(gix)
                pltpu.async_copy(h_hbm.at[gixb[p]], rows.at[p], sem_g)
            else:
                pltpu.async_copy(h_hbm.at[sidx.at[b]], rows.at[p], sem_g)

        def wait_gather():
            pltpu.make_async_copy(h_hbm.at[pl.ds(0, K)], rows.at[0],
                                  sem_g).wait()

        def issue_scatter(b, p):
            pltpu.async_copy(sbuf.at[p], acc.at[didx.at[b]], sem_s,
                             add=True)

        def wait_scatter():
            pltpu.make_async_copy(sbuf.at[0], acc.at[pl.ds(0, K)],
                                  sem_s).wait()

        def scale(b, p, e0sb):
            def scale16(t, carry):
                sv = sidx[b, pl.ds(t * LN, LN)]
                dv = didx[b, pl.ds(t * LN, LN)]
                valid = (e0sb + b * K + t * LN + iota16) < Ereal
                gs = plsc.load_gather(tabs, [sv])
                gd = plsc.load_gather(tabd, [dv])
                al = gs + gd
                al = jnp.maximum(al, 0.2 * al)
                wv = jnp.where(valid, jnp.exp(al), 0.0)
                plsc.addupdate_scatter(dtab, [dv], wv)
                for lane in range(LN):
                    kk = t * LN + lane
                    a0 = wv[lane]
                    v0 = jnp.full((LN,), a0, F32)
                    for q in range(cw // LN):
                        sbuf[p, kk, pl.ds(q * LN, LN)] = (
                            rows[p, kk, pl.ds(q * LN, LN)] * v0)
                return carry

            def scale16_pl(t):
                scale16(t, 0)
            plsc.parallel_loop(0, nvec, unroll=3)(scale16_pl)

        def chunk_body(j, carry):
            ci = c * cpc + j
            for z in range(rpt // zr):
                pltpu.sync_copy(zbuf, acc.at[pl.ds(s * rpt + z * zr, zr)])

            def dzero(r, carry2):
                dtab[pl.ds(r * LN, LN)] = zero16
                return carry2
            lax.fori_loop(0, Nn // LN, dzero, 0)
            plsc.subcore_barrier()
            if n_chunks > 1:
                pltpu.sync_copy(as_hbm.at[ci], tabs)
                pltpu.sync_copy(ad_hbm.at[ci], tabd)
            else:
                pltpu.sync_copy(as_hbm, tabs)
                pltpu.sync_copy(ad_hbm, tabd)

            def sb_body(q, carry2):
                row0 = brow0 + q * nb_sb
                pltpu.sync_copy(src_hbm.at[pl.ds(row0, nb_sb)], sidx)
                pltpu.sync_copy(dst_hbm.at[pl.ds(row0, nb_sb)], didx)
                e0sb = ebase + q * SB
                issue_gather(0, 0, ci)
                issue_gather(1, 1, ci)
                for p in range(2):
                    wait_gather()
                    scale(p, p, e0sb)
                    issue_scatter(p, p)
                    issue_gather(p + 2, p, ci)

                def pair(g, carry3):
                    for p in range(2):
                        b = 2 * g + p
                        wait_gather()
                        wait_scatter()
                        scale(b, p, e0sb)
                        issue_scatter(b, p)
                        issue_gather(b + 2, p, ci)
                    return carry3
                lax.fori_loop(1, nbp - 1, pair, 0)
                for p in range(2):
                    b = nb_sb - 2 + p
                    wait_gather()
                    wait_scatter()
                    scale(b, p, e0sb)
                    issue_scatter(b, p)
                wait_scatter()
                wait_scatter()
                return carry2
            lax.fori_loop(0, nsb, sb_body, 0)
            pltpu.sync_copy(dtab, dstage.at[s])
            plsc.subcore_barrier()
            for i in range(NS):
                off = i * 640
                sz = 640 if i < NS - 1 else Nn - 640 * (NS - 1)

                def dwork(off=off, sz=sz):
                    pltpu.sync_copy(dstage.at[0, pl.ds(off, sz)],
                                    dsum.at[pl.ds(0, sz)])

                    def dred(r, carry2):
                        pltpu.sync_copy(dstage.at[r, pl.ds(off, sz)],
                                        dstrip.at[pl.ds(0, sz)])

                        def dacc(v, carry3):
                            dsum[pl.ds(v * LN, LN)] = (
                                dsum[pl.ds(v * LN, LN)] +
                                dstrip[pl.ds(v * LN, LN)])
                            return carry3
                        lax.fori_loop(0, sz // LN, dacc, 0)
                        return carry2
                    lax.fori_loop(1, NS, dred, 0)
                    pltpu.sync_copy(dsum.at[pl.ds(0, sz)],
                                    den_hbm.at[ci, pl.ds(off, sz)])
                pl.when(s == i)(dwork)
            pltpu.sync_copy(acc.at[pl.ds(s * rpt, rpt)],
                            out_hbm.at[ci, pl.ds(s * rpt, rpt)])
            plsc.subcore_barrier()
            return carry
        lax.fori_loop(0, cpc, chunk_body, 0)

    return k


def _gat_tables(asrc, adst, n_chunks):
    Nn, H = asrc.shape
    hpc = H // n_chunks
    if n_chunks > 1:
        f = lambda a: (a.reshape(Nn, n_chunks, hpc)
                       .transpose(1, 0, 2).reshape(n_chunks, hpc * Nn))
        return f(asrc), f(adst)
    return asrc.reshape(Nn * H), adst.reshape(Nn * H)


def kernel(x, edge_index, W1, as1, ad1, b1, W2, as2, ad2, b2,
           W3, as3, ad3, b3, cW1, cb1, cW2, cb2):
    Nn, F = x.shape
    E = edge_index.shape[1]
    Ereal = E + Nn
    EP = ((Ereal + 165888 - 1) // 165888) * 165888
    loop = jnp.arange(Nn, dtype=I32)
    pad = jnp.zeros((EP - Ereal,), I32)
    src = jnp.concatenate([edge_index[0].astype(I32), loop, pad])
    dst = jnp.concatenate([edge_index[1].astype(I32), loop, pad])
    src12, dst12 = src.reshape(-1, 96), dst.reshape(-1, 96)
    src3, dst3 = src.reshape(-1, 64), dst.reshape(-1, 64)

    H, C = 8, 64
    HC = H * C
    S8 = (jnp.arange(HC)[:, None] // C == jnp.arange(H)[None, :]).astype(F32)
    S1 = jnp.ones((C, 1), F32)

    h4, asrc, adst = _tc_in(x, W1, as1.reshape(1, HC), ad1.reshape(1, HC),
                            S8, n_chunks=8)
    ts, td = _gat_tables(asrc, adst, 8)
    sc12 = _sc_edge_kernel(Nn, EP, Ereal, n_chunks=8, cpc=4, cw=64,
                           K=96, SB=5184, epw=EP // NS,
                           split_edges=False, tabw=Nn)
    acc1, den1 = sc12(h4.reshape(8 * Nn, 64), src12, dst12, ts, td)
    den1 = den1.T

    h4, asrc, adst = _tc_mid(acc1, den1, b1.reshape(1, HC), W2,
                             as2.reshape(1, HC), ad2.reshape(1, HC), S8,
                             n_chunks=8, cw=64, out_chunks=8)
    ts, td = _gat_tables(asrc, adst, 8)
    acc2, den2 = sc12(h4.reshape(8 * Nn, 64), src12, dst12, ts, td)
    den2 = den2.T

    h1, asrc, adst = _tc_mid(acc2, den2, b2.reshape(1, HC), W3,
                             as3.reshape(1, C), ad3.reshape(1, C), S1,
                             n_chunks=8, cw=64, out_chunks=1)
    ts, td = _gat_tables(asrc, adst, 1)
    sc3 = _sc_edge_kernel(Nn, EP, Ereal, n_chunks=1, cpc=1, cw=64,
                          K=64, SB=3456, epw=EP // (NC * NS),
                          split_edges=True, tabw=Nn)
    acc3, den3 = sc3(h1.reshape(Nn, 64), src3, dst3, ts, td)
    den3 = den3.T

    return _tc_final(acc3, den3, b3.reshape(1, C), cW1, cb1.reshape(1, -1),
                     cW2, cb2.reshape(1, -1), cw=64)
